# baseline (speedup 1.0000x reference)
import functools

import jax
import jax.numpy as jnp
from jax import lax
from jax.experimental import pallas as pl
from jax.experimental.pallas import tpu as pltpu
from jax.experimental.pallas import tpu_sc as plsc

NC = 2
NS = 16
NW = NC * NS

LANES = 16
NBUF = 4
NIDX = 8


def _combine_matmul_body(p_ref, w_ref, b_ref, o_ref):
    s = p_ref[0] + p_ref[1]
    o_ref[...] = jnp.dot(s, w_ref[...],
                         preferred_element_type=jnp.float32) + b_ref[...]


def _make_spmm(n, d, e, chunk):
    epw = e // NW
    nchunks = epw // chunk
    assert nchunks >= 8 and (nchunks - 5) % NIDX == 0
    main = nchunks - 2
    rpt = (n // NS) // 8 * 8
    tail = n - NS * rpt

    mesh = plsc.VectorSubcoreMesh(core_axis_name="c", subcore_axis_name="s")

    @functools.partial(
        pl.kernel,
        mesh=mesh,
        out_type=jax.ShapeDtypeStruct((NC, n, d), jnp.float32),
        scratch_types=[
            pltpu.VMEM((NIDX, 2, chunk), jnp.int32),
            pltpu.VMEM((NIDX, 1, chunk), jnp.float32),
            pltpu.VMEM((chunk, d), jnp.float32),
            pltpu.VMEM((chunk, d), jnp.float32),
            pltpu.VMEM((chunk, d), jnp.float32),
            pltpu.VMEM((chunk, d), jnp.float32),
            pltpu.VMEM_SHARED((n, d), jnp.float32),
            [pltpu.SemaphoreType.DMA] * NIDX,
            [pltpu.SemaphoreType.DMA] * NBUF,
            [pltpu.SemaphoreType.DMA] * NBUF,
        ],
    )
    def spmm(h_hbm, row_hbm, col_hbm, ew_hbm, out_hbm,
             idx_v, w_v, buf0, buf1, buf2, buf3, acc_sh,
             isems, gsems, ssems):
        cid = lax.axis_index("c")
        sid = lax.axis_index("s")
        wid = cid * NS + sid
        bufs = (buf0, buf1, buf2, buf3)

        zeros = jnp.zeros((LANES,), jnp.float32)
        for j in range(chunk):
            for d0 in range(0, d, LANES):
                buf0[j, pl.ds(d0, LANES)] = zeros
        zbase = sid * rpt
        nfull, rem = divmod(rpt, chunk)
        for t in range(nfull):
            pltpu.sync_copy(buf0,
                            acc_sh.at[pl.ds(zbase + t * chunk, chunk)])
        if rem:
            pltpu.sync_copy(buf0.at[pl.ds(0, rem)],
                            acc_sh.at[pl.ds(zbase + nfull * chunk, rem)])
        if tail:
            @pl.when(sid == 0)
            def _zero_tail():
                pltpu.sync_copy(buf0.at[pl.ds(0, tail)],
                                acc_sh.at[pl.ds(NS * rpt, tail)])
        plsc.subcore_barrier()

        def start_idx(c, p):
            pltpu.async_copy(row_hbm.at[wid, c], idx_v.at[p, pl.ds(0, 1)],
                             isems[p])
            pltpu.async_copy(col_hbm.at[wid, c], idx_v.at[p, pl.ds(1, 1)],
                             isems[p])
            pltpu.async_copy(ew_hbm.at[wid, c], w_v.at[p], isems[p])

        def wait_idx(c, p):
            pltpu.make_async_copy(row_hbm.at[wid, c],
                                  idx_v.at[p, pl.ds(0, 1)], isems[p]).wait()
            pltpu.make_async_copy(col_hbm.at[wid, c],
                                  idx_v.at[p, pl.ds(1, 1)], isems[p]).wait()
            pltpu.make_async_copy(ew_hbm.at[wid, c], w_v.at[p],
                                  isems[p]).wait()

        def start_gather(p, b):
            pltpu.async_copy(h_hbm.at[idx_v.at[p, 1]], bufs[b], gsems[b])

        def wait_gather(p, b):
            pltpu.make_async_copy(h_hbm.at[idx_v.at[p, 1]], bufs[b],
                                  gsems[b]).wait()

        def start_scatter(p, b):
            pltpu.async_copy(bufs[b], acc_sh.at[idx_v.at[p, 0]],
                             ssems[b], add=True)

        def wait_scatter(p, b):
            pltpu.make_async_copy(bufs[b], acc_sh.at[idx_v.at[p, 0]],
                                  ssems[b]).wait()

        def scale(p, b):
            buf = bufs[b]

            @pl.loop(0, chunk // LANES)
            def _groups(g):
                wv = w_v[p, 0, pl.ds(g * LANES, LANES)]
                for l in range(LANES):
                    ws = wv[l]
                    j = g * LANES + l
                    for d0 in range(0, d, LANES):
                        sl = pl.ds(d0, LANES)
                        buf[j, sl] = buf[j, sl] * ws

        def emit_chunk(c, p, b, *, wait_prev, prefetch, gather_next):
            if wait_prev:
                wait_scatter((p + 5) % NIDX, (b + 1) % NBUF)
            if prefetch:
                start_idx(c + 2, (p + 2) % NIDX)
            if gather_next:
                wait_idx(c + 1, (p + 1) % NIDX)
                start_gather((p + 1) % NIDX, (b + 1) % NBUF)
            wait_gather(p, b)
            scale(p, b)
            start_scatter(p, b)

        start_idx(0, 0)
        start_idx(1, 1)
        wait_idx(0, 0)
        start_gather(0, 0)
        emit_chunk(0, 0, 0, wait_prev=False, prefetch=True, gather_next=True)
        emit_chunk(1, 1, 1, wait_prev=False, prefetch=True, gather_next=True)
        emit_chunk(2, 2, 2, wait_prev=False, prefetch=True, gather_next=True)

        @pl.loop(3, main, step=NIDX)
        def _main(k):
            for j in range(NIDX):
                emit_chunk(k + j, (3 + j) % NIDX, (3 + j) % NBUF,
                           wait_prev=True, prefetch=True, gather_next=True)

        for c in range(main, nchunks):
            emit_chunk(c, c % NIDX, c % NBUF, wait_prev=True, prefetch=False,
                       gather_next=c + 1 < nchunks)
        for c in range(nchunks - 3, nchunks):
            wait_scatter(c % NIDX, c % NBUF)

        plsc.subcore_barrier()
        pltpu.sync_copy(acc_sh.at[pl.ds(zbase, rpt)],
                        out_hbm.at[cid, pl.ds(zbase, rpt)])
        if tail:
            @pl.when(sid == 0)
            def _write_tail():
                pltpu.sync_copy(acc_sh.at[pl.ds(NS * rpt, tail)],
                                out_hbm.at[cid, pl.ds(NS * rpt, tail)])

    return spmm


def kernel(x, edge_index, edge_weight, W, b):
    n, d_in = x.shape
    d_out = W.shape[1]
    e = edge_weight.shape[0]

    chunk = 80
    epw = e // NW
    nchunks = epw // chunk
    row = edge_index[0].reshape(NW, nchunks, 1, chunk)
    col = edge_index[1].reshape(NW, nchunks, 1, chunk)
    ew = edge_weight.reshape(NW, nchunks, 1, chunk)

    spmm = _make_spmm(n, d_in, e, chunk)
    partials = spmm(x, row, col, ew)

    blk = 1000
    out = pl.pallas_call(
        _combine_matmul_body,
        grid=(n // blk,),
        in_specs=[
            pl.BlockSpec((NC, blk, d_in), lambda i: (0, i, 0)),
            pl.BlockSpec((d_in, d_out), lambda i: (0, 0)),
            pl.BlockSpec((1, d_out), lambda i: (0, 0)),
        ],
        out_specs=pl.BlockSpec((blk, d_out), lambda i: (i, 0)),
        out_shape=jax.ShapeDtypeStruct((n, d_out), jnp.float32),
    )(partials, W, b.reshape(1, d_out))
    return out


# speedup vs baseline: 1.0078x; 1.0078x over previous
import functools

import jax
import jax.numpy as jnp
from jax import lax
from jax.experimental import pallas as pl
from jax.experimental.pallas import tpu as pltpu
from jax.experimental.pallas import tpu_sc as plsc

NC = 2
NS = 16
NW = NC * NS

LANES = 16
NBUF = 3
NIDX = 6


def _combine_matmul_body(p_ref, w_ref, b_ref, o_ref):
    s = p_ref[0] + p_ref[1]
    o_ref[...] = jnp.dot(s, w_ref[...],
                         preferred_element_type=jnp.float32) + b_ref[...]


def _make_spmm(n, d, e, chunk):
    epw = e // NW
    nchunks = epw // chunk
    assert nchunks >= 6 and (nchunks - 5) % NIDX == 0
    main = nchunks - 2
    rpt = (n // NS) // 8 * 8
    tail = n - NS * rpt

    mesh = plsc.VectorSubcoreMesh(core_axis_name="c", subcore_axis_name="s")

    @functools.partial(
        pl.kernel,
        mesh=mesh,
        out_type=jax.ShapeDtypeStruct((NC, n, d), jnp.float32),
        scratch_types=[
            pltpu.VMEM((NIDX, 2, chunk), jnp.int32),
            pltpu.VMEM((NIDX, 1, chunk), jnp.float32),
            pltpu.VMEM((chunk, d), jnp.float32),
            pltpu.VMEM((chunk, d), jnp.float32),
            pltpu.VMEM((chunk, d), jnp.float32),
            pltpu.VMEM_SHARED((n, d), jnp.float32),
            [pltpu.SemaphoreType.DMA] * NIDX,
            [pltpu.SemaphoreType.DMA] * NBUF,
            [pltpu.SemaphoreType.DMA] * NBUF,
        ],
    )
    def spmm(h_hbm, row_hbm, col_hbm, ew_hbm, out_hbm,
             idx_v, w_v, buf0, buf1, buf2, acc_sh,
             isems, gsems, ssems):
        cid = lax.axis_index("c")
        sid = lax.axis_index("s")
        wid = cid * NS + sid
        bufs = (buf0, buf1, buf2)

        zeros = jnp.zeros((LANES,), jnp.float32)
        for j in range(chunk):
            for d0 in range(0, d, LANES):
                buf0[j, pl.ds(d0, LANES)] = zeros
        zbase = sid * rpt
        nfull, rem = divmod(rpt, chunk)
        for t in range(nfull):
            pltpu.sync_copy(buf0,
                            acc_sh.at[pl.ds(zbase + t * chunk, chunk)])
        if rem:
            pltpu.sync_copy(buf0.at[pl.ds(0, rem)],
                            acc_sh.at[pl.ds(zbase + nfull * chunk, rem)])
        if tail:
            @pl.when(sid == 0)
            def _zero_tail():
                pltpu.sync_copy(buf0.at[pl.ds(0, tail)],
                                acc_sh.at[pl.ds(NS * rpt, tail)])
        plsc.subcore_barrier()

        def start_idx(c, p):
            pltpu.async_copy(row_hbm.at[wid, c], idx_v.at[p, pl.ds(0, 1)],
                             isems[p])
            pltpu.async_copy(col_hbm.at[wid, c], idx_v.at[p, pl.ds(1, 1)],
                             isems[p])
            pltpu.async_copy(ew_hbm.at[wid, c], w_v.at[p], isems[p])

        def wait_idx(c, p):
            pltpu.make_async_copy(row_hbm.at[wid, c],
                                  idx_v.at[p, pl.ds(0, 1)], isems[p]).wait()
            pltpu.make_async_copy(col_hbm.at[wid, c],
                                  idx_v.at[p, pl.ds(1, 1)], isems[p]).wait()
            pltpu.make_async_copy(ew_hbm.at[wid, c], w_v.at[p],
                                  isems[p]).wait()

        def start_gather(p, b):
            pltpu.async_copy(h_hbm.at[idx_v.at[p, 1]], bufs[b], gsems[b])

        def wait_gather(p, b):
            pltpu.make_async_copy(h_hbm.at[idx_v.at[p, 1]], bufs[b],
                                  gsems[b]).wait()

        def start_scatter(p, b):
            pltpu.async_copy(bufs[b], acc_sh.at[idx_v.at[p, 0]],
                             ssems[b], add=True)

        def wait_scatter(p, b):
            pltpu.make_async_copy(bufs[b], acc_sh.at[idx_v.at[p, 0]],
                                  ssems[b]).wait()

        def scale(p, b):
            buf = bufs[b]

            @pl.loop(0, chunk // LANES)
            def _groups(g):
                wv = w_v[p, 0, pl.ds(g * LANES, LANES)]
                for l in range(LANES):
                    ws = wv[l]
                    j = g * LANES + l
                    for d0 in range(0, d, LANES):
                        sl = pl.ds(d0, LANES)
                        buf[j, sl] = buf[j, sl] * ws

        def emit_chunk(c, p, b, *, wait_prev, prefetch, gather_next):
            if wait_prev:
                wait_scatter((p + 4) % NIDX, (b + 1) % NBUF)
            if prefetch:
                start_idx(c + 2, (p + 2) % NIDX)
            if gather_next:
                wait_idx(c + 1, (p + 1) % NIDX)
                start_gather((p + 1) % NIDX, (b + 1) % NBUF)
            wait_gather(p, b)
            scale(p, b)
            start_scatter(p, b)

        start_idx(0, 0)
        start_idx(1, 1)
        wait_idx(0, 0)
        start_gather(0, 0)
        emit_chunk(0, 0, 0, wait_prev=False, prefetch=True, gather_next=True)
        emit_chunk(1, 1, 1, wait_prev=False, prefetch=True, gather_next=True)
        emit_chunk(2, 2, 2, wait_prev=True, prefetch=True, gather_next=True)

        @pl.loop(3, main, step=NIDX)
        def _main(k):
            for j in range(NIDX):
                emit_chunk(k + j, (3 + j) % NIDX, j % NBUF,
                           wait_prev=True, prefetch=True, gather_next=True)

        for c in range(main, nchunks):
            emit_chunk(c, c % NIDX, c % NBUF, wait_prev=True, prefetch=False,
                       gather_next=c + 1 < nchunks)
        wait_scatter((nchunks - 2) % NIDX, (nchunks - 2) % NBUF)
        wait_scatter((nchunks - 1) % NIDX, (nchunks - 1) % NBUF)

        plsc.subcore_barrier()
        pltpu.sync_copy(acc_sh.at[pl.ds(zbase, rpt)],
                        out_hbm.at[cid, pl.ds(zbase, rpt)])
        if tail:
            @pl.when(sid == 0)
            def _write_tail():
                pltpu.sync_copy(acc_sh.at[pl.ds(NS * rpt, tail)],
                                out_hbm.at[cid, pl.ds(NS * rpt, tail)])

    return spmm


def kernel(x, edge_index, edge_weight, W, b):
    n, d_in = x.shape
    d_out = W.shape[1]
    e = edge_weight.shape[0]

    chunk = 80
    epw = e // NW
    nchunks = epw // chunk
    row = edge_index[0].reshape(NW, nchunks, 1, chunk)
    col = edge_index[1].reshape(NW, nchunks, 1, chunk)
    ew = edge_weight.reshape(NW, nchunks, 1, chunk)

    spmm = _make_spmm(n, d_in, e, chunk)
    partials = spmm(x, row, col, ew)

    blk = 1000
    out = pl.pallas_call(
        _combine_matmul_body,
        grid=(n // blk,),
        in_specs=[
            pl.BlockSpec((NC, blk, d_in), lambda i: (0, i, 0)),
            pl.BlockSpec((d_in, d_out), lambda i: (0, 0)),
            pl.BlockSpec((1, d_out), lambda i: (0, 0)),
        ],
        out_specs=pl.BlockSpec((blk, d_out), lambda i: (i, 0)),
        out_shape=jax.ShapeDtypeStruct((n, d_out), jnp.float32),
    )(partials, W, b.reshape(1, d_out))
    return out
